# baseline (speedup 1.0000x reference)
import functools

import jax
import jax.numpy as jnp
from jax import lax
from jax.experimental import pallas as pl
from jax.experimental.pallas import tpu as pltpu
from jax.experimental.pallas import tpu_sc as plsc

FIELD = 26
K = 16
NPAD = 32

_P_W1 = 0
_P_W2 = 50
_P_W3 = 65
_P_B0 = 68
_P_B1 = 78
_P_B2 = 83
_P_B3 = 86
_P_W0SC = 87
_P_LEN = 96

_H0, _H1, _H2 = 10, 5, 3


def _fm_body(idx_hbm, w_hbm, v_hbm, w0t_hbm, par_hbm, out_hbm,
             idx_v, rows_v, wv_v, w0t_v, par_v, out_v, sem_v, sem_w):
    cid = lax.axis_index("c")
    sid = lax.axis_index("s")

    @pl.when(jnp.logical_and(cid == 0, sid == 0))
    def _():
        pltpu.sync_copy(idx_hbm, idx_v)
        cp_v = pltpu.async_copy(v_hbm.at[idx_v], rows_v, sem_v)
        cp_w = pltpu.async_copy(w_hbm.at[idx_v], wv_v, sem_w)
        pltpu.sync_copy(w0t_hbm, w0t_v)
        pltpu.sync_copy(par_hbm, par_v)
        cp_v.wait()
        cp_w.wait()

        pvec = [par_v[pl.ds(c * 16, 16)] for c in range(_P_LEN // 16)]

        def P(i):
            return pvec[i // 16][i % 16]

        s = jnp.zeros((K,), jnp.float32)
        ssq = jnp.float32(0.0)
        acc = [jnp.zeros((K,), jnp.float32) for _ in range(_H0)]
        for i in range(FIELD):
            v = rows_v[i, :]
            s = s + v
            ssq = ssq + jnp.sum(v * v)
            for j in range(_H0):
                acc[j] = acc[j] + v * w0t_v[j, pl.ds(i * K, K)]
        inter = 0.5 * (jnp.sum(s * s) - ssq)

        lane = lax.iota(jnp.int32, 16)
        tail = jnp.where(lane < (FIELD - K), wv_v[pl.ds(K, K)], 0.0)
        wsum = jnp.sum(wv_v[pl.ds(0, K)] + tail)

        h0 = [jnp.maximum(jnp.sum(acc[j]) + P(_P_B0 + j), 0.0)
              for j in range(_H0)]
        h1 = []
        for b in range(_H1):
            t = P(_P_B1 + b)
            for a in range(_H0):
                t = t + h0[a] * P(_P_W1 + a * _H1 + b)
            h1.append(jnp.maximum(t, 0.0))
        h2 = []
        for b in range(_H2):
            t = P(_P_B2 + b)
            for a in range(_H1):
                t = t + h1[a] * P(_P_W2 + a * _H2 + b)
            h2.append(jnp.maximum(t, 0.0))
        dnn = P(_P_B3)
        for a in range(_H2):
            dnn = dnn + h2[a] * P(_P_W3 + a)

        res = inter + wsum + P(_P_W0SC) + dnn
        out_v[...] = jnp.full((16,), res, jnp.float32)
        pltpu.sync_copy(out_v, out_hbm)


_mesh = plsc.VectorSubcoreMesh(core_axis_name="c", subcore_axis_name="s")

_fm_call = functools.partial(
    pl.kernel, mesh=_mesh,
    out_type=jax.ShapeDtypeStruct((16,), jnp.float32),
    compiler_params=pltpu.CompilerParams(
        needs_layout_passes=False, use_tc_tiling_on_sc=False),
    scratch_types=[
        pltpu.VMEM((NPAD,), jnp.int32),
        pltpu.VMEM((NPAD, K), jnp.float32),
        pltpu.VMEM((NPAD,), jnp.float32),
        pltpu.VMEM((_H0, FIELD * K), jnp.float32),
        pltpu.VMEM((_P_LEN,), jnp.float32),
        pltpu.VMEM((16,), jnp.float32),
        pltpu.SemaphoreType.DMA,
        pltpu.SemaphoreType.DMA,
    ],
)(_fm_body)


def kernel(feature, w_table, v_table, w0, W0, b0, W1, b1, W2, b2, W3, b3):
    idx = jnp.concatenate(
        [feature.astype(jnp.int32), jnp.zeros((NPAD - FIELD,), jnp.int32)])
    params = jnp.concatenate([
        W1.reshape(-1), W2.reshape(-1), W3.reshape(-1),
        b0, b1, b2, b3, w0,
        jnp.zeros((_P_LEN - 88,), jnp.float32)])
    out = _fm_call(idx, w_table.reshape(-1), v_table, W0.T, params)
    return out[:1]


# speedup vs baseline: 6.4018x; 6.4018x over previous
import functools

import jax
import jax.numpy as jnp
from jax import lax
from jax.experimental import pallas as pl
from jax.experimental.pallas import tpu as pltpu
from jax.experimental.pallas import tpu_sc as plsc

FIELD = 26
K = 16
NPAD = 32
LANES = 128

_P_W1 = 0
_P_W2 = 50
_P_W3 = 65
_P_B0 = 68
_P_B1 = 78
_P_B2 = 83
_P_B3 = 86
_P_W0SC = 87
_P_LEN = 96

_H0, _H1, _H2 = 10, 5, 3


def _fm_body(idx_hbm, w_hbm, vt_hbm, w0t_hbm, par_hbm, out_hbm,
             idx_v, vblk_v, wblk_v, w0t_v, par_v, out_v, sem_v, sem_w):
    cid = lax.axis_index("c")
    sid = lax.axis_index("s")

    @pl.when(jnp.logical_and(cid == 0, sid == 0))
    def _():
        pltpu.sync_copy(idx_hbm, idx_v)
        iv0 = idx_v[pl.ds(0, 16)]
        iv1 = idx_v[pl.ds(16, 16)]
        fm0 = lax.rem(iv0, LANES)
        fm1 = lax.rem(iv1, LANES)
        fb0 = lax.mul(lax.div(iv0, LANES), LANES)
        fb1 = lax.mul(lax.div(iv1, LANES), LANES)

        copies = []
        for i in range(FIELD):
            base = pl.multiple_of(fb0[i] if i < 16 else fb1[i - 16], LANES)
            copies.append(pltpu.async_copy(
                vt_hbm.at[:, pl.ds(base, LANES)], vblk_v.at[i], sem_v))
            copies.append(pltpu.async_copy(
                w_hbm.at[pl.ds(base, LANES)], wblk_v.at[i], sem_w))
        pltpu.sync_copy(w0t_hbm, w0t_v)
        pltpu.sync_copy(par_hbm, par_v)
        for c in copies:
            c.wait()

        pvec = [par_v[pl.ds(c * 16, 16)] for c in range(_P_LEN // 16)]

        def P(i):
            return pvec[i // 16][i % 16]

        rowidx = lax.iota(jnp.int32, 16)
        s = jnp.zeros((K,), jnp.float32)
        ssq = jnp.float32(0.0)
        acc = [jnp.zeros((K,), jnp.float32) for _ in range(_H0)]
        for i in range(FIELD):
            lane = fm0[i] if i < 16 else fm1[i - 16]
            v = plsc.load_gather(
                vblk_v, [jnp.full((16,), i, jnp.int32), rowidx,
                         jnp.full((16,), lane, jnp.int32)])
            s = s + v
            ssq = ssq + jnp.sum(v * v)
            for j in range(_H0):
                acc[j] = acc[j] + v * w0t_v[pl.ds(j * FIELD * K + i * K, K)]
        inter = 0.5 * (jnp.sum(s * s) - ssq)

        wlo = plsc.load_gather(wblk_v, [rowidx, fm0])
        whi = plsc.load_gather(wblk_v, [rowidx + 16, fm1])
        wsum = jnp.sum(wlo + jnp.where(rowidx < (FIELD - K), whi, 0.0))

        h0 = [jnp.maximum(jnp.sum(acc[j]) + P(_P_B0 + j), 0.0)
              for j in range(_H0)]
        h1 = []
        for b in range(_H1):
            t = P(_P_B1 + b)
            for a in range(_H0):
                t = t + h0[a] * P(_P_W1 + a * _H1 + b)
            h1.append(jnp.maximum(t, 0.0))
        h2 = []
        for b in range(_H2):
            t = P(_P_B2 + b)
            for a in range(_H1):
                t = t + h1[a] * P(_P_W2 + a * _H2 + b)
            h2.append(jnp.maximum(t, 0.0))
        dnn = P(_P_B3)
        for a in range(_H2):
            dnn = dnn + h2[a] * P(_P_W3 + a)

        res = inter + wsum + P(_P_W0SC) + dnn
        out_v[...] = jnp.full((16,), res, jnp.float32)
        pltpu.sync_copy(out_v, out_hbm)


_mesh = plsc.VectorSubcoreMesh(core_axis_name="c", subcore_axis_name="s")

_fm_call = functools.partial(
    pl.kernel, mesh=_mesh,
    out_type=jax.ShapeDtypeStruct((16,), jnp.float32),
    compiler_params=pltpu.CompilerParams(
        needs_layout_passes=False, use_tc_tiling_on_sc=True),
    scratch_types=[
        pltpu.VMEM((NPAD,), jnp.int32),
        pltpu.VMEM((FIELD, 16, LANES), jnp.float32),
        pltpu.VMEM((NPAD, LANES), jnp.float32),
        pltpu.VMEM((_H0 * FIELD * K,), jnp.float32),
        pltpu.VMEM((_P_LEN,), jnp.float32),
        pltpu.VMEM((16,), jnp.float32),
        pltpu.SemaphoreType.DMA,
        pltpu.SemaphoreType.DMA,
    ],
)(_fm_body)


def kernel(feature, w_table, v_table, w0, W0, b0, W1, b1, W2, b2, W3, b3):
    idx = jnp.concatenate(
        [feature.astype(jnp.int32), jnp.zeros((NPAD - FIELD,), jnp.int32)])
    params = jnp.concatenate([
        W1.reshape(-1), W2.reshape(-1), W3.reshape(-1),
        b0, b1, b2, b3, w0,
        jnp.zeros((_P_LEN - 88,), jnp.float32)])
    out = _fm_call(idx, w_table.reshape(-1), v_table.T,
                   W0.T.reshape(-1), params)
    return out[:1]
